# Initial kernel scaffold; baseline (speedup 1.0000x reference)
import functools

import jax
import jax.numpy as jnp
from jax import lax
from jax.experimental import pallas as pl
from jax.experimental.pallas import tpu as pltpu
from jax.experimental.pallas import tpu_sc as plsc

N = 100000
C = 25000
D = 128
H = 64
W = 72
CPAD = 25088
TROWS = 1568
ZR = 196
RS = 6248
NBLK = 50
SPAD = 28672
GPW = 896
GCH = GPW // 128

_mesh = plsc.VectorSubcoreMesh(
    core_axis_name="c", subcore_axis_name="s", num_cores=2, num_subcores=16
)


@functools.partial(
    pl.kernel,
    out_type=(
        jax.ShapeDtypeStruct((CPAD, D), jnp.float32),
        jax.ShapeDtypeStruct((SPAD, 8), jnp.float32),
    ),
    mesh=_mesh,
    scratch_types=dict(
        acc=pltpu.VMEM_SHARED((CPAD, W), jnp.float32),
        xbuf=pltpu.VMEM((128, W), jnp.float32),
        idxb=pltpu.VMEM((128,), jnp.int32),
        idxg=pltpu.VMEM((128,), jnp.int32),
        grows=pltpu.VMEM((128, 8), jnp.float32),
        zbuf=pltpu.VMEM((ZR, W), jnp.float32),
        dbuf=pltpu.VMEM((ZR, W), jnp.float32),
        obuf=pltpu.VMEM((ZR, H), jnp.float32),
    ),
)
def _pool_kernel(x, cm, posb, si, xnew, pbnew, *, acc, xbuf, idxb, idxg,
                 grows, zbuf, dbuf, obuf):
    c = lax.axis_index("c")
    s = lax.axis_index("s")
    wid = s * 2 + c
    iota16 = jnp.arange(16, dtype=jnp.int32)
    zero16 = jnp.zeros((16,), jnp.float32)
    pat = jnp.where(iota16 == 8, 1.0, 0.0).astype(jnp.float32)

    gbase = wid * GPW
    for j in range(GCH):
        pltpu.sync_copy(si.at[pl.ds(gbase + j * 128, 128)], idxg)
        pltpu.sync_copy(posb.at[idxg], grows)
        pltpu.sync_copy(grows, pbnew.at[pl.ds(gbase + j * 128, 128)])

    def zrow(i, carry):
        for j0 in (0, 16, 32, 48, 56):
            zbuf[i, j0:j0 + 16] = zero16
        return carry

    lax.fori_loop(0, ZR, zrow, 0)
    rbase = s * TROWS
    for k in range(8):
        pltpu.sync_copy(zbuf, acc.at[pl.ds(rbase + k * ZR, ZR)])

    def prow(i, carry):
        xbuf[i, 56:72] = pat
        return carry

    lax.fori_loop(0, 128, prow, 0)

    plsc.subcore_barrier()

    tstart = s * RS
    tend = tstart + RS + jnp.where(s == 15, N - 16 * RS, 0)

    def sblock(b, carry):
        base = tstart + b * 128

        @pl.when(base < tend)
        def _():
            base_c = jnp.minimum(base, tend - 128)
            pltpu.sync_copy(
                x.at[pl.ds(base_c, 128), pl.ds(c * H, H)],
                xbuf.at[:, pl.ds(0, H)],
            )
            pltpu.sync_copy(cm.at[pl.ds(base_c, 128)], idxb)
            off = base - base_c
            for j0 in range(0, 128, 16):
                v = idxb[j0:j0 + 16]
                keep = (j0 + iota16) >= off
                idxb[j0:j0 + 16] = jnp.where(keep, v, C + iota16)
            pltpu.sync_copy(xbuf, acc.at[idxb], add=True)

        return carry

    lax.fori_loop(0, NBLK, sblock, 0)

    plsc.subcore_barrier()

    for k in range(8):
        r0 = rbase + k * ZR
        pltpu.sync_copy(acc.at[pl.ds(r0, ZR)], dbuf)

        def drow(i, carry):
            cnt = dbuf[i, 64]
            rec = 1.0 / jnp.maximum(cnt, 1.0)
            for j0 in (0, 16, 32, 48):
                obuf[i, j0:j0 + 16] = dbuf[i, j0:j0 + 16] * rec
            return carry

        lax.fori_loop(0, ZR, drow, 0)
        pltpu.sync_copy(obuf, xnew.at[pl.ds(r0, ZR), pl.ds(c * H, H)])


def kernel(x, pos, batch, cluster_map, sample_index, edge_index):
    xf = jnp.asarray(x, jnp.float32)
    cm = jnp.asarray(cluster_map, jnp.int32)
    si = jnp.asarray(sample_index, jnp.int32)
    bi = jnp.asarray(batch, jnp.int32)
    posb = jnp.concatenate(
        [
            jnp.asarray(pos, jnp.float32),
            lax.bitcast_convert_type(bi, jnp.float32)[:, None],
            jnp.zeros((N, 4), jnp.float32),
        ],
        axis=1,
    )
    si_pad = jnp.concatenate(
        [si, jnp.arange(SPAD - C, dtype=jnp.int32) % N]
    )
    xnew, pbnew = _pool_kernel(xf, cm, posb, si_pad)
    x_new = xnew[:C]
    pos_new = pbnew[:C, 0:3]
    batch_new = lax.bitcast_convert_type(pbnew[:C, 3], jnp.int32)
    return x_new, pos_new, edge_index, batch_new


# speedup vs baseline: 2.0390x; 2.0390x over previous
import functools

import jax
import jax.numpy as jnp
from jax import lax
from jax.experimental import pallas as pl
from jax.experimental.pallas import tpu as pltpu
from jax.experimental.pallas import tpu_sc as plsc

N = 100000
C = 25000
D = 128
H = 64
CPAD = 25088
TROWS = 1568
ZR = 112
RS = 6248
NBLK = 50
SPAD = 28672
GPW = 896
GCH = GPW // 128

_mesh = plsc.VectorSubcoreMesh(
    core_axis_name="c", subcore_axis_name="s", num_cores=2, num_subcores=16
)


@functools.partial(
    pl.kernel,
    out_type=(
        jax.ShapeDtypeStruct((2, CPAD, H), jnp.float32),
        jax.ShapeDtypeStruct((SPAD, 8), jnp.float32),
    ),
    mesh=_mesh,
    compiler_params=pltpu.CompilerParams(use_tc_tiling_on_sc=False),
    scratch_types=dict(
        acc=pltpu.VMEM_SHARED((CPAD, H), jnp.float32),
        cacc=pltpu.VMEM_SHARED((CPAD,), jnp.float32),
        xbuf=pltpu.VMEM((128, H), jnp.float32),
        onebuf=pltpu.VMEM((128,), jnp.float32),
        idxb=pltpu.VMEM((128,), jnp.int32),
        idxg=pltpu.VMEM((128,), jnp.int32),
        grows=pltpu.VMEM((128, 8), jnp.float32),
        dbuf=pltpu.VMEM((ZR, H), jnp.float32),
        cbuf=pltpu.VMEM((ZR,), jnp.float32),
    ),
)
def _pool_kernel(x, cm, posb, si, xnewh, pbnew, *, acc, cacc, xbuf, onebuf,
                 idxb, idxg, grows, dbuf, cbuf):
    c = lax.axis_index("c")
    s = lax.axis_index("s")
    wid = s * 2 + c
    iota16 = jnp.arange(16, dtype=jnp.int32)
    zero16 = jnp.zeros((16,), jnp.float32)
    ones16 = jnp.ones((16,), jnp.float32)

    gbase = wid * GPW
    for j in range(GCH):
        pltpu.sync_copy(si.at[pl.ds(gbase + j * 128, 128)], idxg)
        pltpu.sync_copy(posb.at[idxg], grows)
        pltpu.sync_copy(grows, pbnew.at[pl.ds(gbase + j * 128, 128)])

    def zrow(i, carry):
        for j0 in (0, 16, 32, 48):
            dbuf[i, j0:j0 + 16] = zero16
        return carry

    lax.fori_loop(0, ZR, zrow, 0)
    for j0 in range(0, ZR, 16):
        cbuf[j0:j0 + 16] = zero16
    for j0 in range(0, 128, 16):
        onebuf[j0:j0 + 16] = ones16
    rbase = s * TROWS
    for k in range(14):
        pltpu.sync_copy(dbuf, acc.at[pl.ds(rbase + k * ZR, ZR)])
        pltpu.sync_copy(cbuf, cacc.at[pl.ds(rbase + k * ZR, ZR)])

    plsc.subcore_barrier()

    tstart = s * RS
    tend = tstart + RS + jnp.where(s == 15, N - 16 * RS, 0)

    def sblock(b, carry):
        base = tstart + b * 128

        @pl.when(base < tend)
        def _():
            base_c = jnp.minimum(base, tend - 128)
            pltpu.sync_copy(
                x.at[pl.ds(base_c, 128), pl.ds(c * H, H)], xbuf
            )
            pltpu.sync_copy(cm.at[pl.ds(base_c, 128)], idxb)
            off = base - base_c
            for j0 in range(0, 128, 16):
                v = idxb[j0:j0 + 16]
                keep = (j0 + iota16) >= off
                idxb[j0:j0 + 16] = jnp.where(keep, v, C + iota16)
            pltpu.sync_copy(xbuf, acc.at[idxb], add=True)
            pltpu.sync_copy(onebuf, cacc.at[idxb], add=True)

        return carry

    lax.fori_loop(0, NBLK, sblock, 0)

    plsc.subcore_barrier()

    lane_idx = [jnp.full((16, 1), l, jnp.int32) for l in range(16)]
    dnums = lax.GatherDimensionNumbers(
        offset_dims=(), collapsed_slice_dims=(0,), start_index_map=(0,)
    )
    for k in range(14):
        r0 = rbase + k * ZR
        pltpu.sync_copy(acc.at[pl.ds(r0, ZR)], dbuf)
        pltpu.sync_copy(cacc.at[pl.ds(r0, ZR)], cbuf)

        def dgrp(g, carry):
            cv = cbuf[pl.ds(g * 16, 16)]
            rec = 1.0 / jnp.maximum(cv, 1.0)
            for l in range(16):
                rec_bc = lax.gather(
                    rec, lane_idx[l], dnums, slice_sizes=(1,),
                    mode=lax.GatherScatterMode.PROMISE_IN_BOUNDS,
                )
                i = g * 16 + l
                for j0 in (0, 16, 32, 48):
                    dbuf[i, j0:j0 + 16] = dbuf[i, j0:j0 + 16] * rec_bc
            return carry

        lax.fori_loop(0, ZR // 16, dgrp, 0)
        pltpu.sync_copy(dbuf, xnewh.at[c, pl.ds(r0, ZR)])


def kernel(x, pos, batch, cluster_map, sample_index, edge_index):
    xf = jnp.asarray(x, jnp.float32)
    cm = jnp.asarray(cluster_map, jnp.int32)
    si = jnp.asarray(sample_index, jnp.int32)
    bi = jnp.asarray(batch, jnp.int32)
    posb = jnp.concatenate(
        [
            jnp.asarray(pos, jnp.float32),
            bi.astype(jnp.float32)[:, None],
            jnp.zeros((N, 4), jnp.float32),
        ],
        axis=1,
    )
    si_pad = jnp.concatenate(
        [si, jnp.arange(SPAD - C, dtype=jnp.int32) % N]
    )
    xnewh, pbnew = _pool_kernel(xf, cm, posb, si_pad)
    x_new = jnp.concatenate([xnewh[0, :C], xnewh[1, :C]], axis=1)
    pos_new = pbnew[:C, 0:3]
    batch_new = pbnew[:C, 3].astype(jnp.int32)
    return x_new, pos_new, edge_index, batch_new


# speedup vs baseline: 2.5928x; 1.2716x over previous
import functools

import jax
import jax.numpy as jnp
from jax import lax
from jax.experimental import pallas as pl
from jax.experimental.pallas import tpu as pltpu
from jax.experimental.pallas import tpu_sc as plsc

N = 100000
C = 25000
D = 128
H = 64
CPAD = 25088
TROWS = 1568
ZR = 112
RS = 6248
NBLK = 50
GPW = 784
GCH = 7

_mesh = plsc.VectorSubcoreMesh(
    core_axis_name="c", subcore_axis_name="s", num_cores=2, num_subcores=16
)


@functools.partial(
    pl.kernel,
    out_type=(
        jax.ShapeDtypeStruct((C, D), jnp.float32),
        jax.ShapeDtypeStruct((C, 8), jnp.float32),
    ),
    mesh=_mesh,
    compiler_params=pltpu.CompilerParams(use_tc_tiling_on_sc=False),
    scratch_types=dict(
        acc=pltpu.VMEM_SHARED((CPAD, H), jnp.float32),
        cacc=pltpu.VMEM_SHARED((CPAD,), jnp.float32),
        xbuf=pltpu.VMEM((128, H), jnp.float32),
        onebuf=pltpu.VMEM((128,), jnp.float32),
        idxb=pltpu.VMEM((128,), jnp.int32),
        idxg=pltpu.VMEM((128,), jnp.int32),
        gpos=pltpu.VMEM((128, 8), jnp.float32),
        dbuf=pltpu.VMEM((ZR, H), jnp.float32),
        cbuf=pltpu.VMEM((ZR,), jnp.float32),
    ),
)
def _pool_kernel(x, posb, cm, si, xnew, pbnew, *, acc, cacc,
                 xbuf, onebuf, idxb, idxg, gpos, dbuf, cbuf):
    c = lax.axis_index("c")
    s = lax.axis_index("s")
    wid = s * 2 + c
    iota16 = jnp.arange(16, dtype=jnp.int32)
    zero16 = jnp.zeros((16,), jnp.float32)
    ones16 = jnp.ones((16,), jnp.float32)

    wbase = wid * GPW
    for j in range(GCH):
        gb = jnp.minimum(wbase + j * 128, C - 128)
        pltpu.sync_copy(si.at[pl.ds(gb, 128)], idxg)
        pltpu.sync_copy(posb.at[idxg], gpos)
        pltpu.sync_copy(gpos, pbnew.at[pl.ds(gb, 128)])

    def zrow(i, carry):
        for j0 in (0, 16, 32, 48):
            dbuf[i, j0:j0 + 16] = zero16
        return carry

    lax.fori_loop(0, ZR, zrow, 0)
    for j0 in range(0, ZR, 16):
        cbuf[j0:j0 + 16] = zero16
    for j0 in range(0, 128, 16):
        onebuf[j0:j0 + 16] = ones16
    rbase = s * TROWS
    for k in range(14):
        pltpu.sync_copy(dbuf, acc.at[pl.ds(rbase + k * ZR, ZR)])
        pltpu.sync_copy(cbuf, cacc.at[pl.ds(rbase + k * ZR, ZR)])

    plsc.subcore_barrier()

    tstart = s * RS
    tend = tstart + RS + jnp.where(s == 15, N - 16 * RS, 0)

    def sblock(b, carry):
        base = tstart + b * 128

        @pl.when(base < tend)
        def _():
            base_c = jnp.minimum(base, tend - 128)
            pltpu.sync_copy(
                x.at[pl.ds(base_c, 128), pl.ds(c * H, H)], xbuf
            )
            pltpu.sync_copy(cm.at[pl.ds(base_c, 128)], idxb)
            off = base - base_c
            for j0 in range(0, 128, 16):
                v = idxb[j0:j0 + 16]
                keep = (j0 + iota16) >= off
                idxb[j0:j0 + 16] = jnp.where(keep, v, C + iota16)
            pltpu.sync_copy(xbuf, acc.at[idxb], add=True)
            pltpu.sync_copy(onebuf, cacc.at[idxb], add=True)

        return carry

    lax.fori_loop(0, NBLK, sblock, 0)

    plsc.subcore_barrier()

    lane_idx = [jnp.full((16, 1), l, jnp.int32) for l in range(16)]
    dnums = lax.GatherDimensionNumbers(
        offset_dims=(), collapsed_slice_dims=(0,), start_index_map=(0,)
    )
    for k in range(14):
        r0 = jnp.minimum(rbase + k * ZR, C - ZR)
        pltpu.sync_copy(acc.at[pl.ds(r0, ZR)], dbuf)
        pltpu.sync_copy(cacc.at[pl.ds(r0, ZR)], cbuf)

        def dgrp(g, carry):
            cv = cbuf[pl.ds(g * 16, 16)]
            rec = 1.0 / jnp.maximum(cv, 1.0)
            for l in range(16):
                rec_bc = lax.gather(
                    rec, lane_idx[l], dnums, slice_sizes=(1,),
                    mode=lax.GatherScatterMode.PROMISE_IN_BOUNDS,
                )
                i = g * 16 + l
                for j0 in (0, 16, 32, 48):
                    dbuf[i, j0:j0 + 16] = dbuf[i, j0:j0 + 16] * rec_bc
            return carry

        lax.fori_loop(0, ZR // 16, dgrp, 0)
        pltpu.sync_copy(dbuf, xnew.at[pl.ds(r0, ZR), pl.ds(c * H, H)])


def kernel(x, pos, batch, cluster_map, sample_index, edge_index):
    xf = jnp.asarray(x, jnp.float32)
    bi = jnp.asarray(batch, jnp.int32)
    cm = jnp.asarray(cluster_map, jnp.int32)
    si = jnp.asarray(sample_index, jnp.int32)
    posb = jnp.concatenate(
        [
            jnp.asarray(pos, jnp.float32),
            bi.astype(jnp.float32)[:, None],
            jnp.zeros((N, 4), jnp.float32),
        ],
        axis=1,
    )
    x_new, pbnew = _pool_kernel(xf, posb, cm, si)
    pos_new = pbnew[:, 0:3]
    batch_new = pbnew[:, 3].astype(jnp.int32)
    return x_new, pos_new, edge_index, batch_new


# speedup vs baseline: 2.9066x; 1.1210x over previous
import functools

import jax
import jax.numpy as jnp
from jax import lax
from jax.experimental import pallas as pl
from jax.experimental.pallas import tpu as pltpu
from jax.experimental.pallas import tpu_sc as plsc

N = 100000
C = 25000
D = 128
H = 64
CPAD = 25088
TROWS = 1568
ZR = 64
NZCH = 25
RS = 6248
BR = 112
NBLK = 58
GPW = 784
GCH = 7

_mesh = plsc.VectorSubcoreMesh(
    core_axis_name="c", subcore_axis_name="s", num_cores=2, num_subcores=16
)


@functools.partial(
    pl.kernel,
    out_type=(
        jax.ShapeDtypeStruct((C, D), jnp.float32),
        jax.ShapeDtypeStruct((C, 8), jnp.float32),
    ),
    mesh=_mesh,
    compiler_params=pltpu.CompilerParams(use_tc_tiling_on_sc=False),
    scratch_types=dict(
        acc=pltpu.VMEM_SHARED((CPAD, H), jnp.float32),
        cacc=pltpu.VMEM_SHARED((CPAD,), jnp.float32),
        xbuf=pltpu.VMEM((2, BR, H), jnp.float32),
        idxb=pltpu.VMEM((2, BR), jnp.int32),
        onebuf=pltpu.VMEM((BR,), jnp.float32),
        idxg=pltpu.VMEM((128,), jnp.int32),
        gpos=pltpu.VMEM((128, 8), jnp.float32),
        dbuf=pltpu.VMEM((ZR, H), jnp.float32),
        cbuf=pltpu.VMEM((ZR,), jnp.float32),
        semf0=pltpu.SemaphoreType.DMA,
        semf1=pltpu.SemaphoreType.DMA,
        sems=pltpu.SemaphoreType.DMA,
    ),
)
def _pool_kernel(x, posb, cm, si, xnew, pbnew, *, acc, cacc, xbuf, idxb,
                 onebuf, idxg, gpos, dbuf, cbuf, semf0, semf1, sems):
    c = lax.axis_index("c")
    s = lax.axis_index("s")
    wid = s * 2 + c
    iota16 = jnp.arange(16, dtype=jnp.int32)
    zero16 = jnp.zeros((16,), jnp.float32)
    ones16 = jnp.ones((16,), jnp.float32)
    semf = (semf0, semf1)

    wbase = wid * GPW
    for j in range(GCH):
        gb = jnp.minimum(wbase + j * 128, C - 128)
        pltpu.sync_copy(si.at[pl.ds(gb, 128)], idxg)
        pltpu.sync_copy(posb.at[idxg], gpos)
        pltpu.sync_copy(gpos, pbnew.at[pl.ds(gb, 128)])

    def zrow(i, carry):
        for j0 in (0, 16, 32, 48):
            dbuf[i, j0:j0 + 16] = zero16
        return carry

    lax.fori_loop(0, ZR, zrow, 0)
    for j0 in range(0, ZR, 16):
        cbuf[j0:j0 + 16] = zero16
    for j0 in range(0, BR, 16):
        onebuf[j0:j0 + 16] = ones16
    rbase = s * TROWS
    for k in range(TROWS // ZR):
        pltpu.sync_copy(dbuf, acc.at[pl.ds(rbase + k * ZR, ZR)])
        pltpu.sync_copy(cbuf, cacc.at[pl.ds(rbase + k * ZR, ZR)])
    pltpu.sync_copy(dbuf, acc.at[pl.ds(rbase + TROWS - ZR, ZR)])
    pltpu.sync_copy(cbuf, cacc.at[pl.ds(rbase + TROWS - ZR, ZR)])

    plsc.subcore_barrier()

    tstart = s * RS
    tend = tstart + RS + jnp.where(s == 15, N - 16 * RS, 0)
    hi = tend - BR

    def fetch(p, base_c):
        pltpu.async_copy(
            x.at[pl.ds(base_c, BR), pl.ds(c * H, H)], xbuf.at[p], semf[p]
        )
        pltpu.async_copy(cm.at[pl.ds(base_c, BR)], idxb.at[p], semf[p])

    for p in range(2):
        fetch(p, jnp.minimum(tstart + p * BR, hi))

    def spair(g, carry):
        for p in range(2):
            base = tstart + (g * 2 + p) * BR
            base_c = jnp.minimum(base, hi)
            pltpu.make_async_copy(
                x.at[pl.ds(base_c, BR), pl.ds(c * H, H)], xbuf.at[p], semf[p]
            ).wait()
            pltpu.make_async_copy(
                cm.at[pl.ds(base_c, BR)], idxb.at[p], semf[p]
            ).wait()

            @pl.when(base < tend)
            def _():
                off = base - base_c
                for j0 in range(0, BR, 16):
                    v = idxb[p, j0:j0 + 16]
                    keep = (j0 + iota16) >= off
                    idxb[p, j0:j0 + 16] = jnp.where(keep, v, C + iota16)
                a1 = pltpu.async_copy(
                    xbuf.at[p], acc.at[idxb.at[p]], sems, add=True
                )
                a2 = pltpu.async_copy(
                    onebuf, cacc.at[idxb.at[p]], sems, add=True
                )
                a1.wait()
                a2.wait()

            fetch(p, jnp.minimum(base + 2 * BR, hi))
        return carry

    lax.fori_loop(0, NBLK // 2, spair, 0)
    for p in range(2):
        pltpu.make_async_copy(
            x.at[pl.ds(0, BR), pl.ds(c * H, H)], xbuf.at[p], semf[p]
        ).wait()
        pltpu.make_async_copy(cm.at[pl.ds(0, BR)], idxb.at[p], semf[p]).wait()

    plsc.subcore_barrier()

    lane_idx = [jnp.full((16, 1), l, jnp.int32) for l in range(16)]
    dnums = lax.GatherDimensionNumbers(
        offset_dims=(), collapsed_slice_dims=(0,), start_index_map=(0,)
    )
    for k in range(NZCH):
        r0 = jnp.minimum(rbase + k * ZR, rbase + TROWS - ZR)
        r0 = jnp.minimum(r0, C - ZR)
        pltpu.sync_copy(acc.at[pl.ds(r0, ZR)], dbuf)
        pltpu.sync_copy(cacc.at[pl.ds(r0, ZR)], cbuf)

        def dgrp(g, carry):
            cv = cbuf[pl.ds(g * 16, 16)]
            rec = 1.0 / jnp.maximum(cv, 1.0)
            for l in range(16):
                rec_bc = lax.gather(
                    rec, lane_idx[l], dnums, slice_sizes=(1,),
                    mode=lax.GatherScatterMode.PROMISE_IN_BOUNDS,
                )
                i = g * 16 + l
                for j0 in (0, 16, 32, 48):
                    dbuf[i, j0:j0 + 16] = dbuf[i, j0:j0 + 16] * rec_bc
            return carry

        lax.fori_loop(0, ZR // 16, dgrp, 0)
        pltpu.sync_copy(dbuf, xnew.at[pl.ds(r0, ZR), pl.ds(c * H, H)])


def kernel(x, pos, batch, cluster_map, sample_index, edge_index):
    xf = jnp.asarray(x, jnp.float32)
    bi = jnp.asarray(batch, jnp.int32)
    cm = jnp.asarray(cluster_map, jnp.int32)
    si = jnp.asarray(sample_index, jnp.int32)
    posb = jnp.concatenate(
        [
            jnp.asarray(pos, jnp.float32),
            bi.astype(jnp.float32)[:, None],
            jnp.zeros((N, 4), jnp.float32),
        ],
        axis=1,
    )
    x_new, pbnew = _pool_kernel(xf, posb, cm, si)
    pos_new = pbnew[:, 0:3]
    batch_new = pbnew[:, 3].astype(jnp.int32)
    return x_new, pos_new, edge_index, batch_new


# speedup vs baseline: 3.1623x; 1.0880x over previous
import functools

import jax
import jax.numpy as jnp
from jax import lax
from jax.experimental import pallas as pl
from jax.experimental.pallas import tpu as pltpu
from jax.experimental.pallas import tpu_sc as plsc

N = 100000
C = 25000
D = 128
H = 64
CPAD = 25088
TROWS = 1568
ZR = 64
NZCH = 25
RS = 6248
BR = 112
NBLK = 58
GPW = 784
GCH = 7

_mesh = plsc.VectorSubcoreMesh(
    core_axis_name="c", subcore_axis_name="s", num_cores=2, num_subcores=16
)


@functools.partial(
    pl.kernel,
    out_type=(
        jax.ShapeDtypeStruct((2 * C, H), jnp.float32),
        jax.ShapeDtypeStruct((C, 8), jnp.float32),
    ),
    mesh=_mesh,
    compiler_params=pltpu.CompilerParams(use_tc_tiling_on_sc=False),
    scratch_types=dict(
        acc=pltpu.VMEM_SHARED((CPAD, H), jnp.float32),
        cacc=pltpu.VMEM_SHARED((CPAD,), jnp.float32),
        xbuf=pltpu.VMEM((2, BR, H), jnp.float32),
        idxb=pltpu.VMEM((2, BR), jnp.int32),
        idxr=pltpu.VMEM((2, BR), jnp.int32),
        idxw=pltpu.VMEM((ZR,), jnp.int32),
        onebuf=pltpu.VMEM((BR,), jnp.float32),
        idxg=pltpu.VMEM((128,), jnp.int32),
        gpos=pltpu.VMEM((128, 8), jnp.float32),
        dbuf=pltpu.VMEM((ZR, H), jnp.float32),
        cbuf=pltpu.VMEM((ZR,), jnp.float32),
        semf0=pltpu.SemaphoreType.DMA,
        semf1=pltpu.SemaphoreType.DMA,
        sems=pltpu.SemaphoreType.DMA,
    ),
)
def _pool_kernel(x2, posb, cm, si, xnew2, pbnew, *, acc, cacc, xbuf, idxb,
                 idxr, idxw, onebuf, idxg, gpos, dbuf, cbuf, semf0, semf1,
                 sems):
    c = lax.axis_index("c")
    s = lax.axis_index("s")
    wid = s * 2 + c
    iota16 = jnp.arange(16, dtype=jnp.int32)
    zero16 = jnp.zeros((16,), jnp.float32)
    ones16 = jnp.ones((16,), jnp.float32)
    semf = (semf0, semf1)

    wbase = wid * GPW
    for j in range(GCH):
        gb = jnp.minimum(wbase + j * 128, C - 128)
        pltpu.sync_copy(si.at[pl.ds(gb, 128)], idxg)
        pltpu.sync_copy(posb.at[idxg], gpos)
        pltpu.sync_copy(gpos, pbnew.at[pl.ds(gb, 128)])

    def zrow(i, carry):
        for j0 in (0, 16, 32, 48):
            dbuf[i, j0:j0 + 16] = zero16
        return carry

    lax.fori_loop(0, ZR, zrow, 0)
    for j0 in range(0, ZR, 16):
        cbuf[j0:j0 + 16] = zero16
    for j0 in range(0, BR, 16):
        onebuf[j0:j0 + 16] = ones16
    rbase = s * TROWS
    for k in range(TROWS // ZR):
        pltpu.sync_copy(dbuf, acc.at[pl.ds(rbase + k * ZR, ZR)])
        pltpu.sync_copy(cbuf, cacc.at[pl.ds(rbase + k * ZR, ZR)])
    pltpu.sync_copy(dbuf, acc.at[pl.ds(rbase + TROWS - ZR, ZR)])
    pltpu.sync_copy(cbuf, cacc.at[pl.ds(rbase + TROWS - ZR, ZR)])

    plsc.subcore_barrier()

    tstart = s * RS
    tend = tstart + RS + jnp.where(s == 15, N - 16 * RS, 0)
    hi = tend - BR

    def fetch(p, base_c):
        for j0 in range(0, BR, 16):
            idxr[p, j0:j0 + 16] = 2 * (base_c + j0 + iota16) + c
        pltpu.async_copy(x2.at[idxr.at[p]], xbuf.at[p], semf[p])
        pltpu.async_copy(cm.at[pl.ds(base_c, BR)], idxb.at[p], semf[p])

    for p in range(2):
        fetch(p, jnp.minimum(tstart + p * BR, hi))

    def spair(g, carry):
        for p in range(2):
            base = tstart + (g * 2 + p) * BR
            base_c = jnp.minimum(base, hi)
            pltpu.make_async_copy(
                x2.at[pl.ds(0, BR)], xbuf.at[p], semf[p]
            ).wait()
            pltpu.make_async_copy(
                cm.at[pl.ds(base_c, BR)], idxb.at[p], semf[p]
            ).wait()

            @pl.when(base < tend)
            def _():
                off = base - base_c
                for j0 in range(0, BR, 16):
                    v = idxb[p, j0:j0 + 16]
                    keep = (j0 + iota16) >= off
                    idxb[p, j0:j0 + 16] = jnp.where(keep, v, C + iota16)
                a1 = pltpu.async_copy(
                    xbuf.at[p], acc.at[idxb.at[p]], sems, add=True
                )
                a2 = pltpu.async_copy(
                    onebuf, cacc.at[idxb.at[p]], sems, add=True
                )
                a1.wait()
                a2.wait()

            fetch(p, jnp.minimum(base + 2 * BR, hi))
        return carry

    lax.fori_loop(0, NBLK // 2, spair, 0)
    for p in range(2):
        pltpu.make_async_copy(x2.at[pl.ds(0, BR)], xbuf.at[p], semf[p]).wait()
        pltpu.make_async_copy(cm.at[pl.ds(0, BR)], idxb.at[p], semf[p]).wait()

    plsc.subcore_barrier()

    lane_idx = [jnp.full((16, 1), l, jnp.int32) for l in range(16)]
    dnums = lax.GatherDimensionNumbers(
        offset_dims=(), collapsed_slice_dims=(0,), start_index_map=(0,)
    )
    for k in range(NZCH):
        r0 = jnp.minimum(rbase + k * ZR, rbase + TROWS - ZR)
        r0 = jnp.minimum(r0, C - ZR)
        pltpu.sync_copy(acc.at[pl.ds(r0, ZR)], dbuf)
        pltpu.sync_copy(cacc.at[pl.ds(r0, ZR)], cbuf)

        def dgrp(g, carry):
            cv = cbuf[pl.ds(g * 16, 16)]
            rec = 1.0 / jnp.maximum(cv, 1.0)
            for l in range(16):
                rec_bc = lax.gather(
                    rec, lane_idx[l], dnums, slice_sizes=(1,),
                    mode=lax.GatherScatterMode.PROMISE_IN_BOUNDS,
                )
                i = g * 16 + l
                for j0 in (0, 16, 32, 48):
                    dbuf[i, j0:j0 + 16] = dbuf[i, j0:j0 + 16] * rec_bc
            return carry

        lax.fori_loop(0, ZR // 16, dgrp, 0)
        for j0 in range(0, ZR, 16):
            idxw[j0:j0 + 16] = 2 * (r0 + j0 + iota16) + c
        pltpu.sync_copy(dbuf, xnew2.at[idxw])


def kernel(x, pos, batch, cluster_map, sample_index, edge_index):
    xf = jnp.asarray(x, jnp.float32)
    bi = jnp.asarray(batch, jnp.int32)
    cm = jnp.asarray(cluster_map, jnp.int32)
    si = jnp.asarray(sample_index, jnp.int32)
    posb = jnp.concatenate(
        [
            jnp.asarray(pos, jnp.float32),
            bi.astype(jnp.float32)[:, None],
            jnp.zeros((N, 4), jnp.float32),
        ],
        axis=1,
    )
    xnew2, pbnew = _pool_kernel(xf.reshape(2 * N, H), posb, cm, si)
    x_new = xnew2.reshape(C, D)
    pos_new = pbnew[:, 0:3]
    batch_new = pbnew[:, 3].astype(jnp.int32)
    return x_new, pos_new, edge_index, batch_new


# speedup vs baseline: 5.1856x; 1.6398x over previous
import functools

import jax
import jax.numpy as jnp
from jax import lax
from jax.experimental import pallas as pl
from jax.experimental.pallas import tpu as pltpu
from jax.experimental.pallas import tpu_sc as plsc

N = 100000
C = 25000
D = 128
H = 64
CPAD = 25088
TROWS = 1568
ZR = 64
NZCH = 25
RS = 6248
BR = 112
NBLK = 58
GPW = 784
GCH = 7

_mesh = plsc.VectorSubcoreMesh(
    core_axis_name="c", subcore_axis_name="s", num_cores=2, num_subcores=16
)


@functools.partial(
    pl.kernel,
    out_type=(
        jax.ShapeDtypeStruct((2 * C, H), jnp.float32),
        jax.ShapeDtypeStruct((C,), jnp.float32),
        jax.ShapeDtypeStruct((C,), jnp.float32),
        jax.ShapeDtypeStruct((C,), jnp.float32),
        jax.ShapeDtypeStruct((C,), jnp.int32),
    ),
    mesh=_mesh,
    compiler_params=pltpu.CompilerParams(use_tc_tiling_on_sc=False),
    scratch_types=dict(
        acc=pltpu.VMEM_SHARED((CPAD, H), jnp.float32),
        cacc=pltpu.VMEM_SHARED((CPAD,), jnp.float32),
        xbuf=pltpu.VMEM((2, BR, H), jnp.float32),
        idxb=pltpu.VMEM((2, BR), jnp.int32),
        idxr=pltpu.VMEM((2, BR), jnp.int32),
        idxw=pltpu.VMEM((ZR,), jnp.int32),
        onebuf=pltpu.VMEM((BR,), jnp.float32),
        idxg=pltpu.VMEM((128,), jnp.int32),
        g0=pltpu.VMEM((128,), jnp.float32),
        g1=pltpu.VMEM((128,), jnp.float32),
        g2=pltpu.VMEM((128,), jnp.float32),
        gb4=pltpu.VMEM((128,), jnp.int32),
        semg=pltpu.SemaphoreType.DMA,
        dbuf=pltpu.VMEM((ZR, H), jnp.float32),
        cbuf=pltpu.VMEM((ZR,), jnp.float32),
        semf0=pltpu.SemaphoreType.DMA,
        semf1=pltpu.SemaphoreType.DMA,
        sems=pltpu.SemaphoreType.DMA,
    ),
)
def _pool_kernel(x2, px, py, pz, bt, cm, si, xnew2, p0new, p1new, p2new,
                 bnew, *, acc, cacc, xbuf, idxb, idxr, idxw, onebuf, idxg,
                 g0, g1, g2, gb4, semg, dbuf, cbuf, semf0, semf1, sems):
    c = lax.axis_index("c")
    s = lax.axis_index("s")
    wid = s * 2 + c
    iota16 = jnp.arange(16, dtype=jnp.int32)
    zero16 = jnp.zeros((16,), jnp.float32)
    ones16 = jnp.ones((16,), jnp.float32)
    semf = (semf0, semf1)

    wbase = wid * GPW
    for j in range(GCH):
        gb = jnp.minimum(wbase + j * 128, C - 128)
        pltpu.sync_copy(si.at[pl.ds(gb, 128)], idxg)
        ds = [
            pltpu.async_copy(px.at[idxg], g0, semg),
            pltpu.async_copy(py.at[idxg], g1, semg),
            pltpu.async_copy(pz.at[idxg], g2, semg),
            pltpu.async_copy(bt.at[idxg], gb4, semg),
        ]
        for d in ds:
            d.wait()
        ds = [
            pltpu.async_copy(g0, p0new.at[pl.ds(gb, 128)], semg),
            pltpu.async_copy(g1, p1new.at[pl.ds(gb, 128)], semg),
            pltpu.async_copy(g2, p2new.at[pl.ds(gb, 128)], semg),
            pltpu.async_copy(gb4, bnew.at[pl.ds(gb, 128)], semg),
        ]
        for d in ds:
            d.wait()

    def zrow(i, carry):
        for j0 in (0, 16, 32, 48):
            dbuf[i, j0:j0 + 16] = zero16
        return carry

    lax.fori_loop(0, ZR, zrow, 0)
    for j0 in range(0, ZR, 16):
        cbuf[j0:j0 + 16] = zero16
    for j0 in range(0, BR, 16):
        onebuf[j0:j0 + 16] = ones16
    rbase = s * TROWS
    for k in range(TROWS // ZR):
        pltpu.sync_copy(dbuf, acc.at[pl.ds(rbase + k * ZR, ZR)])
        pltpu.sync_copy(cbuf, cacc.at[pl.ds(rbase + k * ZR, ZR)])
    pltpu.sync_copy(dbuf, acc.at[pl.ds(rbase + TROWS - ZR, ZR)])
    pltpu.sync_copy(cbuf, cacc.at[pl.ds(rbase + TROWS - ZR, ZR)])

    plsc.subcore_barrier()

    tstart = s * RS
    tend = tstart + RS + jnp.where(s == 15, N - 16 * RS, 0)
    hi = tend - BR

    def fetch(p, base_c):
        for j0 in range(0, BR, 16):
            idxr[p, j0:j0 + 16] = 2 * (base_c + j0 + iota16) + c
        pltpu.async_copy(x2.at[idxr.at[p]], xbuf.at[p], semf[p])
        pltpu.async_copy(cm.at[pl.ds(base_c, BR)], idxb.at[p], semf[p])

    for p in range(2):
        fetch(p, jnp.minimum(tstart + p * BR, hi))

    def spair(g, carry):
        for p in range(2):
            base = tstart + (g * 2 + p) * BR
            base_c = jnp.minimum(base, hi)
            pltpu.make_async_copy(
                x2.at[pl.ds(0, BR)], xbuf.at[p], semf[p]
            ).wait()
            pltpu.make_async_copy(
                cm.at[pl.ds(base_c, BR)], idxb.at[p], semf[p]
            ).wait()

            @pl.when(base < tend)
            def _():
                off = base - base_c
                for j0 in range(0, BR, 16):
                    v = idxb[p, j0:j0 + 16]
                    keep = (j0 + iota16) >= off
                    idxb[p, j0:j0 + 16] = jnp.where(keep, v, C + iota16)
                a1 = pltpu.async_copy(
                    xbuf.at[p], acc.at[idxb.at[p]], sems, add=True
                )
                a2 = pltpu.async_copy(
                    onebuf, cacc.at[idxb.at[p]], sems, add=True
                )
                a1.wait()
                a2.wait()

            fetch(p, jnp.minimum(base + 2 * BR, hi))
        return carry

    lax.fori_loop(0, NBLK // 2, spair, 0)
    for p in range(2):
        pltpu.make_async_copy(x2.at[pl.ds(0, BR)], xbuf.at[p], semf[p]).wait()
        pltpu.make_async_copy(cm.at[pl.ds(0, BR)], idxb.at[p], semf[p]).wait()

    plsc.subcore_barrier()

    lane_idx = [jnp.full((16, 1), l, jnp.int32) for l in range(16)]
    dnums = lax.GatherDimensionNumbers(
        offset_dims=(), collapsed_slice_dims=(0,), start_index_map=(0,)
    )
    for k in range(NZCH):
        r0 = jnp.minimum(rbase + k * ZR, rbase + TROWS - ZR)
        r0 = jnp.minimum(r0, C - ZR)
        pltpu.sync_copy(acc.at[pl.ds(r0, ZR)], dbuf)
        pltpu.sync_copy(cacc.at[pl.ds(r0, ZR)], cbuf)

        def dgrp(g, carry):
            cv = cbuf[pl.ds(g * 16, 16)]
            rec = 1.0 / jnp.maximum(cv, 1.0)
            for l in range(16):
                rec_bc = lax.gather(
                    rec, lane_idx[l], dnums, slice_sizes=(1,),
                    mode=lax.GatherScatterMode.PROMISE_IN_BOUNDS,
                )
                i = g * 16 + l
                for j0 in (0, 16, 32, 48):
                    dbuf[i, j0:j0 + 16] = dbuf[i, j0:j0 + 16] * rec_bc
            return carry

        lax.fori_loop(0, ZR // 16, dgrp, 0)
        for j0 in range(0, ZR, 16):
            idxw[j0:j0 + 16] = 2 * (r0 + j0 + iota16) + c
        pltpu.sync_copy(dbuf, xnew2.at[idxw])


def kernel(x, pos, batch, cluster_map, sample_index, edge_index):
    xf = jnp.asarray(x, jnp.float32)
    pf = jnp.asarray(pos, jnp.float32)
    bi = jnp.asarray(batch, jnp.int32)
    cm = jnp.asarray(cluster_map, jnp.int32)
    si = jnp.asarray(sample_index, jnp.int32)
    xnew2, p0, p1, p2, batch_new = _pool_kernel(
        xf.reshape(2 * N, H), pf[:, 0], pf[:, 1], pf[:, 2], bi, cm, si
    )
    x_new = xnew2.reshape(C, D)
    pos_new = jnp.stack([p0, p1, p2], axis=1)
    return x_new, pos_new, edge_index, batch_new
